# Initial kernel scaffold; baseline (speedup 1.0000x reference)
import jax
import jax.numpy as jnp
from jax.experimental import pallas as pl


def kernel(input, m1, W1, b1, W2, b2):
    raise NotImplementedError("write your pallas kernel here")


# speedup vs baseline: 1.0221x; 1.0221x over previous
import functools

import jax
import jax.numpy as jnp
from jax import lax
from jax.experimental import pallas as pl
from jax.experimental.pallas import tpu as pltpu
from jax.experimental.pallas import tpu_sc as plsc

B, C, N = 4, 384, 576
M, HD = 8192, 256
T = B * N
MB = 512
NMB = M // MB

_NC, _NS = 2, 16
_NW = _NC * _NS
_TPW = T // _NW


def _argmin_body(x_ref, w1_ref, b1_ref, m1t_ref, emb_ref, idx_ref,
                 els_ref, best_ref, bidx_ref):
    j = pl.program_id(0)

    @pl.when(j == 0)
    def _init():
        emb = jnp.dot(x_ref[...], w1_ref[...],
                      preferred_element_type=jnp.float32) + b1_ref[...]
        emb = jnp.maximum(emb, 0.0)
        emb_ref[...] = emb
        emx = jnp.max(emb, axis=1, keepdims=True)
        sh = emb - emx
        els_ref[...] = sh - jnp.log(jnp.sum(jnp.exp(sh), axis=1,
                                            keepdims=True))
        best_ref[...] = jnp.full((T, 1), jnp.inf, jnp.float32)
        bidx_ref[...] = jnp.zeros((T, 1), jnp.int32)

    mb = m1t_ref[...]
    mx = jnp.max(mb, axis=0, keepdims=True)
    e = jnp.exp(mb - mx)
    s = jnp.sum(e, axis=0, keepdims=True)
    p = e / s
    ent = jnp.sum(p * jnp.log(p), axis=0, keepdims=True)

    cross = jnp.dot(els_ref[...], p,
                    preferred_element_type=jnp.float32)
    kl = ent - cross
    bmin = jnp.min(kl, axis=1, keepdims=True)
    lane = lax.broadcasted_iota(jnp.int32, kl.shape, 1)
    barg = jnp.min(jnp.where(kl == bmin, lane, MB), axis=1,
                   keepdims=True) + j * MB
    upd = bmin < best_ref[...]
    bidx_ref[...] = jnp.where(upd, barg, bidx_ref[...])
    best_ref[...] = jnp.where(upd, bmin, best_ref[...])

    @pl.when(j == NMB - 1)
    def _fin():
        idx_ref[...] = bidx_ref[...]


_argmin_call = pl.pallas_call(
    _argmin_body,
    grid=(NMB,),
    in_specs=[
        pl.BlockSpec((T, C), lambda j: (0, 0)),
        pl.BlockSpec((C, HD), lambda j: (0, 0)),
        pl.BlockSpec((1, HD), lambda j: (0, 0)),
        pl.BlockSpec((HD, MB), lambda j: (0, j)),
    ],
    out_specs=[
        pl.BlockSpec((T, HD), lambda j: (0, 0)),
        pl.BlockSpec((T, 1), lambda j: (0, 0)),
    ],
    out_shape=[
        jax.ShapeDtypeStruct((T, HD), jnp.float32),
        jax.ShapeDtypeStruct((T, 1), jnp.int32),
    ],
    scratch_shapes=[
        pltpu.VMEM((T, HD), jnp.float32),
        pltpu.VMEM((T, 1), jnp.float32),
        pltpu.VMEM((T, 1), jnp.int32),
    ],
)


@functools.cache
def _make_gather_sc():
    @functools.partial(
        pl.kernel,
        mesh=plsc.VectorSubcoreMesh(core_axis_name="c", subcore_axis_name="s"),
        out_type=jax.ShapeDtypeStruct((T, HD), jnp.float32),
        scratch_types=[
            pltpu.VMEM((_TPW,), jnp.int32),
            pltpu.VMEM((_TPW, HD), jnp.float32),
            pltpu.SemaphoreType.DMA,
        ],
    )
    def _gather_body(table_hbm, idx_hbm, out_hbm, idx_v, rows_v, sem):
        wid = lax.axis_index("s") * _NC + lax.axis_index("c")
        base = wid * _TPW
        pltpu.sync_copy(idx_hbm.at[pl.ds(base, _TPW)], idx_v)
        pltpu.async_copy(table_hbm.at[idx_v], rows_v, sem).wait()
        pltpu.sync_copy(rows_v, out_hbm.at[pl.ds(base, _TPW)])

    return _gather_body


def _gather_sc(table, idx):
    return _make_gather_sc()(table, idx)


def _out_body(emb_ref, g_ref, w2t_ref, b2_ref, out_ref):
    emb = emb_ref[...]
    cat = jnp.concatenate([emb, emb - g_ref[...]], axis=1)
    out_ref[...] = jnp.dot(cat, w2t_ref[...],
                           preferred_element_type=jnp.float32) + b2_ref[...]


_out_call = pl.pallas_call(
    _out_body,
    out_shape=jax.ShapeDtypeStruct((T, HD), jnp.float32),
)


def kernel(input, m1, W1, b1, W2, b2):
    x_t = input[..., 0].transpose(0, 2, 1).reshape(T, C)
    emb, idx = _argmin_call(x_t, W1.T, b1.reshape(1, HD), m1.T)
    g = _gather_sc(m1, idx.reshape(T))
    out_t = _out_call(emb, g, W2.T, b2.reshape(1, HD))
    return out_t.reshape(B, N, HD).transpose(0, 2, 1)[..., None]


# speedup vs baseline: 1.1336x; 1.1091x over previous
import functools

import jax
import jax.numpy as jnp
from jax import lax
from jax.experimental import pallas as pl
from jax.experimental.pallas import tpu as pltpu
from jax.experimental.pallas import tpu_sc as plsc

B, C, N = 4, 384, 576
M, HD = 8192, 256
T = B * N
MB = 512
NMB = M // MB

_NC, _NS = 2, 16
_NW = _NC * _NS
_TPW = T // _NW


def _argmin_body(x_ref, w1_ref, b1_ref, m1_ref, emb_ref, idx_ref,
                 els_ref, best_ref, bidx_ref):
    j = pl.program_id(0)

    @pl.when(j == 0)
    def _init():
        for b in range(B):
            e_b = jnp.dot(w1_ref[...], x_ref[b],
                          preferred_element_type=jnp.float32) + b1_ref[...]
            e_b = jnp.maximum(e_b, 0.0)
            emb_ref[b] = e_b
            et = jnp.transpose(e_b)
            emx = jnp.max(et, axis=1, keepdims=True)
            sh = et - emx
            els_ref[pl.ds(b * N, N), :] = sh - jnp.log(
                jnp.sum(jnp.exp(sh), axis=1, keepdims=True))
        best_ref[...] = jnp.full((1, T), jnp.inf, jnp.float32)
        bidx_ref[...] = jnp.zeros((1, T), jnp.int32)

    mb = m1_ref[...]
    mx = jnp.max(mb, axis=1, keepdims=True)
    e = jnp.exp(mb - mx)
    s = jnp.sum(e, axis=1, keepdims=True)
    p = e / s
    ent = jnp.sum(p * jnp.log(p), axis=1, keepdims=True)

    cross = lax.dot_general(p, els_ref[...], (((1,), (1,)), ((), ())),
                            preferred_element_type=jnp.float32)
    kl = ent - cross
    bmin = jnp.min(kl, axis=0, keepdims=True)
    barg = jnp.argmin(kl, axis=0).astype(jnp.int32).reshape(1, T) + j * MB
    upd = bmin < best_ref[...]
    bidx_ref[...] = jnp.where(upd, barg, bidx_ref[...])
    best_ref[...] = jnp.where(upd, bmin, best_ref[...])

    @pl.when(j == NMB - 1)
    def _fin():
        idx_ref[...] = bidx_ref[...]


_argmin_call = pl.pallas_call(
    _argmin_body,
    grid=(NMB,),
    in_specs=[
        pl.BlockSpec((B, C, N), lambda j: (0, 0, 0)),
        pl.BlockSpec((HD, C), lambda j: (0, 0)),
        pl.BlockSpec((HD, 1), lambda j: (0, 0)),
        pl.BlockSpec((MB, HD), lambda j: (j, 0)),
    ],
    out_specs=[
        pl.BlockSpec((B, HD, N), lambda j: (0, 0, 0)),
        pl.BlockSpec((1, T), lambda j: (0, 0)),
    ],
    out_shape=[
        jax.ShapeDtypeStruct((B, HD, N), jnp.float32),
        jax.ShapeDtypeStruct((1, T), jnp.int32),
    ],
    scratch_shapes=[
        pltpu.VMEM((T, HD), jnp.float32),
        pltpu.VMEM((1, T), jnp.float32),
        pltpu.VMEM((1, T), jnp.int32),
    ],
)


@functools.cache
def _make_gather_sc():
    @functools.partial(
        pl.kernel,
        mesh=plsc.VectorSubcoreMesh(core_axis_name="c", subcore_axis_name="s"),
        out_type=jax.ShapeDtypeStruct((T, HD), jnp.float32),
        scratch_types=[
            pltpu.VMEM((_TPW,), jnp.int32),
            pltpu.VMEM((_TPW, HD), jnp.float32),
            pltpu.SemaphoreType.DMA,
        ],
    )
    def _gather_body(table_hbm, idx_hbm, out_hbm, idx_v, rows_v, sem):
        wid = lax.axis_index("s") * _NC + lax.axis_index("c")
        base = wid * _TPW
        pltpu.sync_copy(idx_hbm.at[pl.ds(base, _TPW)], idx_v)
        pltpu.async_copy(table_hbm.at[idx_v], rows_v, sem).wait()
        pltpu.sync_copy(rows_v, out_hbm.at[pl.ds(base, _TPW)])

    return _gather_body


def _gather_sc(table, idx):
    return _make_gather_sc()(table, idx)


def _out_body(emb_ref, g_ref, w2_ref, b2_ref, out_ref):
    for b in range(B):
        gt = jnp.transpose(g_ref[pl.ds(b * N, N), :])
        e_b = emb_ref[b]
        cat = jnp.concatenate([e_b, e_b - gt], axis=0)
        out_ref[b] = jnp.dot(w2_ref[...], cat,
                             preferred_element_type=jnp.float32) + b2_ref[...]


_out_call = pl.pallas_call(
    _out_body,
    out_shape=jax.ShapeDtypeStruct((B, HD, N), jnp.float32),
)


def kernel(input, m1, W1, b1, W2, b2):
    x = input[..., 0]
    emb, idx = _argmin_call(x, W1, b1.reshape(HD, 1), m1)
    g = _gather_sc(m1, idx.reshape(T))
    out = _out_call(emb, g, W2, b2.reshape(HD, 1))
    return out[..., None]


# speedup vs baseline: 1.2135x; 1.0705x over previous
import functools

import jax
import jax.numpy as jnp
from jax import lax
from jax.experimental import pallas as pl
from jax.experimental.pallas import tpu as pltpu
from jax.experimental.pallas import tpu_sc as plsc

B, C, N = 4, 384, 576
M, HD = 8192, 256
T = B * N
MB = 2048
NMB = M // MB

_NC, _NS = 2, 16
_NW = _NC * _NS
_TPW = T // _NW


def _argmin_body(x_ref, w1_ref, b1_ref, m1_ref, emb_ref, idx_ref,
                 els_ref, best_ref, bidx_ref):
    j = pl.program_id(0)

    @pl.when(j == 0)
    def _init():
        for b in range(B):
            e_b = jnp.dot(w1_ref[...], x_ref[b],
                          preferred_element_type=jnp.float32) + b1_ref[...]
            e_b = jnp.maximum(e_b, 0.0)
            et = jnp.transpose(e_b)
            emb_ref[pl.ds(b * N, N), :] = et
            emx = jnp.max(et, axis=1, keepdims=True)
            sh = et - emx
            els_ref[pl.ds(b * N, N), :] = sh - jnp.log(
                jnp.sum(jnp.exp(sh), axis=1, keepdims=True))
        best_ref[...] = jnp.full((1, T), jnp.inf, jnp.float32)
        bidx_ref[...] = jnp.zeros((1, T), jnp.int32)

    mb = m1_ref[...]
    mx = jnp.max(mb, axis=1, keepdims=True)
    e = jnp.exp(mb - mx)
    s = jnp.sum(e, axis=1, keepdims=True)
    p = e / s
    ent = jnp.sum(p * jnp.log(p), axis=1, keepdims=True)

    cross = lax.dot_general(p, els_ref[...], (((1,), (1,)), ((), ())),
                            preferred_element_type=jnp.float32)
    kl = ent - cross
    bmin = jnp.min(kl, axis=0, keepdims=True)
    barg = jnp.argmin(kl, axis=0).astype(jnp.int32).reshape(1, T) + j * MB
    upd = bmin < best_ref[...]
    bidx_ref[...] = jnp.where(upd, barg, bidx_ref[...])
    best_ref[...] = jnp.where(upd, bmin, best_ref[...])

    @pl.when(j == NMB - 1)
    def _fin():
        idx_ref[...] = bidx_ref[...]


_argmin_call = pl.pallas_call(
    _argmin_body,
    grid=(NMB,),
    in_specs=[
        pl.BlockSpec((B, C, N), lambda j: (0, 0, 0)),
        pl.BlockSpec((HD, C), lambda j: (0, 0)),
        pl.BlockSpec((HD, 1), lambda j: (0, 0)),
        pl.BlockSpec((MB, HD), lambda j: (j, 0)),
    ],
    out_specs=[
        pl.BlockSpec((T, HD), lambda j: (0, 0)),
        pl.BlockSpec((1, T), lambda j: (0, 0)),
    ],
    out_shape=[
        jax.ShapeDtypeStruct((T, HD), jnp.float32),
        jax.ShapeDtypeStruct((1, T), jnp.int32),
    ],
    scratch_shapes=[
        pltpu.VMEM((T, HD), jnp.float32),
        pltpu.VMEM((1, T), jnp.float32),
        pltpu.VMEM((1, T), jnp.int32),
    ],
)


@functools.cache
def _make_gather_sc():
    @functools.partial(
        pl.kernel,
        mesh=plsc.VectorSubcoreMesh(core_axis_name="c", subcore_axis_name="s"),
        out_type=jax.ShapeDtypeStruct((T, HD), jnp.float32),
        scratch_types=[
            pltpu.VMEM((_TPW,), jnp.int32),
            pltpu.VMEM((_TPW, HD), jnp.float32),
            pltpu.SemaphoreType.DMA,
        ],
    )
    def _gather_body(table_hbm, idx_hbm, out_hbm, idx_v, rows_v, sem):
        wid = lax.axis_index("s") * _NC + lax.axis_index("c")
        base = wid * _TPW
        pltpu.sync_copy(idx_hbm.at[pl.ds(base, _TPW)], idx_v)
        pltpu.async_copy(table_hbm.at[idx_v], rows_v, sem).wait()
        pltpu.sync_copy(rows_v, out_hbm.at[pl.ds(base, _TPW)])

    return _gather_body


def _gather_sc(table, idx):
    return _make_gather_sc()(table, idx)


def _out_body(emb_ref, g_ref, w2t_ref, b2_ref, out_ref):
    e_b = emb_ref[...]
    cat = jnp.concatenate([e_b, e_b - g_ref[...]], axis=1)
    out_ref[...] = jnp.dot(cat, w2t_ref[...],
                           preferred_element_type=jnp.float32) + b2_ref[...]


_out_call = pl.pallas_call(
    _out_body,
    grid=(B,),
    in_specs=[
        pl.BlockSpec((N, HD), lambda b: (b, 0)),
        pl.BlockSpec((N, HD), lambda b: (b, 0)),
        pl.BlockSpec((2 * HD, HD), lambda b: (0, 0)),
        pl.BlockSpec((1, HD), lambda b: (0, 0)),
    ],
    out_specs=pl.BlockSpec((N, HD), lambda b: (b, 0)),
    out_shape=jax.ShapeDtypeStruct((T, HD), jnp.float32),
)


def kernel(input, m1, W1, b1, W2, b2):
    x = input[..., 0]
    emb_t, idx = _argmin_call(x, W1, b1.reshape(HD, 1), m1)
    g = _gather_sc(m1, idx.reshape(T))
    out_t = _out_call(emb_t, g, W2.T, b2.reshape(1, HD))
    return out_t.reshape(B, N, HD).transpose(0, 2, 1)[..., None]
